# Initial kernel scaffold; baseline (speedup 1.0000x reference)
import functools

import jax
import jax.numpy as jnp
from jax import lax
from jax.experimental import pallas as pl
from jax.experimental.pallas import tpu as pltpu
from jax.experimental.pallas import tpu_sc as plsc

N = 10000
D = 128
C = 128
E = 320000

NC = 2
NS = 16
NW = NC * NS

CH = 128
J = -(-E // (NW * CH))
E_PAD = NW * J * CH
R = 632
N_PAD = NS * R

B = 1000


def _sc_aggregate(x, src_slab, dst_slab, zrow, zdeg):
    mesh = plsc.VectorSubcoreMesh(core_axis_name="c", subcore_axis_name="s")

    @functools.partial(
        pl.kernel,
        out_type=(
            jax.ShapeDtypeStruct((NC, N_PAD, D), jnp.float32),
            jax.ShapeDtypeStruct((NC, N_PAD), jnp.float32),
        ),
        mesh=mesh,
        scratch_types=[
            pltpu.VMEM((J, CH), jnp.int32),
            pltpu.VMEM((J, CH), jnp.int32),
            pltpu.VMEM((CH, D), jnp.float32),
            pltpu.VMEM((CH,), jnp.float32),
            pltpu.VMEM_SHARED((N_PAD, D), jnp.float32),
            pltpu.VMEM_SHARED((N_PAD,), jnp.float32),
            pltpu.SemaphoreType.DMA,
        ],
    )
    def k(x_hbm, src_hbm, dst_hbm, zrow_hbm, zdeg_hbm, agg_out, deg_out,
          src_v, dst_v, rows_v, ones_v, agg_s, deg_s, sem):
        c = lax.axis_index("c")
        s = lax.axis_index("s")
        wid = s * NC + c
        pltpu.sync_copy(src_hbm.at[wid], src_v)
        pltpu.sync_copy(dst_hbm.at[wid], dst_v)
        pltpu.sync_copy(zrow_hbm.at[pl.ds(s * R, R)], agg_s.at[pl.ds(s * R, R)])
        pltpu.sync_copy(zdeg_hbm.at[pl.ds(s * R, R)], deg_s.at[pl.ds(s * R, R)])
        for i in range(CH // 16):
            ones_v[pl.ds(i * 16, 16)] = jnp.ones((16,), jnp.float32)
        plsc.subcore_barrier()

        def chunk(j, carry):
            pltpu.async_copy(x_hbm.at[src_v.at[j]], rows_v, sem).wait()
            pltpu.sync_copy(rows_v, agg_s.at[dst_v.at[j]], add=True)
            pltpu.sync_copy(ones_v, deg_s.at[dst_v.at[j]], add=True)
            return carry

        lax.fori_loop(0, J, chunk, 0)
        plsc.subcore_barrier()
        pltpu.sync_copy(agg_s.at[pl.ds(s * R, R)], agg_out.at[c, pl.ds(s * R, R)])
        pltpu.sync_copy(deg_s.at[pl.ds(s * R, R)], deg_out.at[c, pl.ds(s * R, R)])

    return k(x, src_slab, dst_slab, zrow, zdeg)


def _tc_body(nd_ref, x_ref, agg_ref, deg_ref, wsT_ref, b_ref, wnT_ref, out_ref):
    i = pl.program_id(0)
    rows = i * B + lax.broadcasted_iota(jnp.int32, (B, 1), 0)
    mask = rows < nd_ref[0]
    x_blk = jnp.where(mask, x_ref[...], 0.0)
    agg = agg_ref[0] + agg_ref[1]
    deg = deg_ref[0] + deg_ref[1]
    h_neigh = jnp.where(mask, agg / jnp.maximum(deg, 1.0), 0.0)
    acc = jnp.dot(x_blk, wsT_ref[...], preferred_element_type=jnp.float32)
    acc = acc + jnp.dot(h_neigh, wnT_ref[...], preferred_element_type=jnp.float32)
    out_ref[...] = jnp.maximum(acc + b_ref[...], 0.0)


def _tc_matmul(nd, x, agg2, deg3, W_self, b_self, W_neigh):
    return pl.pallas_call(
        _tc_body,
        grid=(N // B,),
        in_specs=[
            pl.BlockSpec(memory_space=pltpu.SMEM),
            pl.BlockSpec((B, D), lambda i: (i, 0)),
            pl.BlockSpec((NC, B, D), lambda i: (0, i, 0)),
            pl.BlockSpec((NC, B, 1), lambda i: (0, i, 0)),
            pl.BlockSpec((D, C), lambda i: (0, 0)),
            pl.BlockSpec((1, C), lambda i: (0, 0)),
            pl.BlockSpec((D, C), lambda i: (0, 0)),
        ],
        out_specs=pl.BlockSpec((B, C), lambda i: (i, 0)),
        out_shape=jax.ShapeDtypeStruct((N, C), jnp.float32),
    )(nd, x, agg2, deg3, W_self.T, b_self.reshape(1, C), W_neigh.T)


def kernel(x, edge_index, num_dst, W_self, b_self, W_neigh):
    src = edge_index[0]
    dst = edge_index[1]
    pad = E_PAD - E
    src_slab = jnp.concatenate(
        [src, jnp.zeros((pad,), jnp.int32)]).reshape(NW, J, CH)
    dst_slab = jnp.concatenate(
        [dst, jnp.full((pad,), N, jnp.int32)]).reshape(NW, J, CH)
    zrow = jnp.zeros((N_PAD, D), jnp.float32)
    zdeg = jnp.zeros((N_PAD,), jnp.float32)
    agg2, deg2 = _sc_aggregate(x, src_slab, dst_slab, zrow, zdeg)
    deg3 = deg2.reshape(NC, N_PAD, 1)
    nd = jnp.asarray(num_dst, jnp.int32).reshape(1)
    return _tc_matmul(nd, x, agg2, deg3, W_self, b_self, W_neigh)


# speedup vs baseline: 5.3266x; 5.3266x over previous
import functools

import jax
import jax.numpy as jnp
from jax import lax
from jax.experimental import pallas as pl
from jax.experimental.pallas import tpu as pltpu
from jax.experimental.pallas import tpu_sc as plsc

N = 10000
D = 128
C = 128
E = 320000

NC = 2
NS = 16
NW = NC * NS

CH = 128
J = -(-E // (NW * CH))
E_PAD = NW * J * CH
R = 640
N_PAD = NS * R

B = 1000


def _sc_aggregate(x, src_slab, dst_slab):
    mesh = plsc.VectorSubcoreMesh(core_axis_name="c", subcore_axis_name="s")

    @functools.partial(
        pl.kernel,
        out_type=(
            jax.ShapeDtypeStruct((NC, N_PAD, D), jnp.float32),
            jax.ShapeDtypeStruct((NC * N_PAD,), jnp.float32),
        ),
        mesh=mesh,
        scratch_types=[
            pltpu.VMEM((J, CH), jnp.int32),
            pltpu.VMEM((J, CH), jnp.int32),
            pltpu.VMEM((CH, D), jnp.float32),
            pltpu.VMEM((CH,), jnp.float32),
            pltpu.VMEM((R,), jnp.float32),
            pltpu.VMEM_SHARED((N_PAD, D), jnp.float32),
            pltpu.VMEM_SHARED((N_PAD,), jnp.float32),
            pltpu.SemaphoreType.DMA,
        ],
    )
    def k(x_hbm, src_hbm, dst_hbm, agg_out, deg_out,
          src_v, dst_v, rows_v, ones_v, deg_tile, agg_s, deg_s, sem):
        c = lax.axis_index("c")
        s = lax.axis_index("s")
        wid = s * NC + c
        pltpu.sync_copy(src_hbm.at[wid], src_v)
        pltpu.sync_copy(dst_hbm.at[wid], dst_v)
        def zero_row(j, carry):
            for i in range(D // 16):
                rows_v[j, pl.ds(i * 16, 16)] = jnp.zeros((16,), jnp.float32)
            return carry

        lax.fori_loop(0, CH, zero_row, 0)
        for k_ in range(R // CH):
            pltpu.sync_copy(rows_v, agg_s.at[pl.ds(s * R + k_ * CH, CH)])

        def zero_deg(j, carry):
            deg_tile[pl.ds(j * 16, 16)] = jnp.zeros((16,), jnp.float32)
            return carry

        lax.fori_loop(0, R // 16, zero_deg, 0)
        pltpu.sync_copy(deg_tile, deg_s.at[pl.ds(s * R, R)])
        for i in range(CH // 16):
            ones_v[pl.ds(i * 16, 16)] = jnp.ones((16,), jnp.float32)
        plsc.subcore_barrier()

        def chunk(j, carry):
            pltpu.async_copy(x_hbm.at[src_v.at[j]], rows_v, sem).wait()
            pltpu.sync_copy(rows_v, agg_s.at[dst_v.at[j]], add=True)
            pltpu.sync_copy(ones_v, deg_s.at[dst_v.at[j]], add=True)
            return carry

        lax.fori_loop(0, J, chunk, 0)
        plsc.subcore_barrier()
        pltpu.sync_copy(agg_s.at[pl.ds(s * R, R)], agg_out.at[c, pl.ds(s * R, R)])
        pltpu.sync_copy(deg_s.at[pl.ds(s * R, R)], deg_tile)
        pltpu.sync_copy(deg_tile, deg_out.at[pl.ds(c * N_PAD + s * R, R)])

    return k(x, src_slab, dst_slab)


def _tc_body(nd_ref, x_ref, agg_ref, deg_ref, wsT_ref, b_ref, wnT_ref, out_ref):
    i = pl.program_id(0)
    rows = i * B + lax.broadcasted_iota(jnp.int32, (B, 1), 0)
    mask = rows < nd_ref[0]
    x_blk = jnp.where(mask, x_ref[...], 0.0)
    agg = agg_ref[0] + agg_ref[1]
    deg = deg_ref[0] + deg_ref[1]
    h_neigh = jnp.where(mask, agg / jnp.maximum(deg, 1.0), 0.0)
    acc = jnp.dot(x_blk, wsT_ref[...], preferred_element_type=jnp.float32)
    acc = acc + jnp.dot(h_neigh, wnT_ref[...], preferred_element_type=jnp.float32)
    out_ref[...] = jnp.maximum(acc + b_ref[...], 0.0)


def _tc_matmul(nd, x, agg2, deg3, W_self, b_self, W_neigh):
    return pl.pallas_call(
        _tc_body,
        grid=(N // B,),
        in_specs=[
            pl.BlockSpec(memory_space=pltpu.SMEM),
            pl.BlockSpec((B, D), lambda i: (i, 0)),
            pl.BlockSpec((NC, B, D), lambda i: (0, i, 0)),
            pl.BlockSpec((NC, B, 1), lambda i: (0, i, 0)),
            pl.BlockSpec((D, C), lambda i: (0, 0)),
            pl.BlockSpec((1, C), lambda i: (0, 0)),
            pl.BlockSpec((D, C), lambda i: (0, 0)),
        ],
        out_specs=pl.BlockSpec((B, C), lambda i: (i, 0)),
        out_shape=jax.ShapeDtypeStruct((N, C), jnp.float32),
    )(nd, x, agg2, deg3, W_self.T, b_self.reshape(1, C), W_neigh.T)


def kernel(x, edge_index, num_dst, W_self, b_self, W_neigh):
    src = edge_index[0]
    dst = edge_index[1]
    pad = E_PAD - E
    src_slab = jnp.concatenate(
        [src, jnp.zeros((pad,), jnp.int32)]).reshape(NW, J, CH)
    dst_slab = jnp.concatenate(
        [dst, jnp.full((pad,), N, jnp.int32)]).reshape(NW, J, CH)
    agg2, deg2 = _sc_aggregate(x, src_slab, dst_slab)
    deg3 = deg2.reshape(NC, N_PAD, 1)
    nd = jnp.asarray(num_dst, jnp.int32).reshape(1)
    return _tc_matmul(nd, x, agg2, deg3, W_self, b_self, W_neigh)
